# baseline (speedup 1.0000x reference)
import jax
import jax.numpy as jnp
from jax import lax
from jax.experimental import pallas as pl
from jax.experimental.pallas import tpu as pltpu
from jax.experimental.pallas import tpu_sc as plsc

N_RAYS = 1048576
RAY_DIM = 16
NUM_VIEWS = 1000
TABLE_COLS = 12

_NC, _NS, _L = 2, 16, 16
_NW = _NC * _NS
_ROWS_PER_W = N_RAYS // _NW
_CHUNK = 2048
_N_CHUNKS = _ROWS_PER_W // _CHUNK


def _sc_body(rays_t_hbm, table_hbm, out9_t_hbm, out3_t_hbm,
             ids_v, table_v, col_v, isem0, isem1, osem0, osem1):
    wid = lax.axis_index("s") * _NC + lax.axis_index("c")
    base_w = wid * _ROWS_PER_W
    isems = (isem0, isem1)
    osems = (osem0, osem1)
    pltpu.sync_copy(table_hbm, table_v)

    def in_copy(h):
        return pltpu.make_async_copy(
            rays_t_hbm.at[pl.ds(RAY_DIM - 2, 1),
                          pl.ds(base_w + h * (_ROWS_PER_W // 2),
                                _ROWS_PER_W // 2)],
            ids_v.at[pl.ds(h, 1), pl.ds(0, _ROWS_PER_W // 2)],
            isems[h])

    def out_copies(ci, b):
        base = base_w + ci * _CHUNK
        cps = [
            pltpu.make_async_copy(
                col_v.at[pl.ds(b * 16, 8), pl.ds(0, _CHUNK)],
                out9_t_hbm.at[pl.ds(0, 8), pl.ds(base, _CHUNK)],
                osems[b]),
            pltpu.make_async_copy(
                col_v.at[pl.ds(b * 16 + 8, 1), pl.ds(0, _CHUNK)],
                out9_t_hbm.at[pl.ds(8, 1), pl.ds(base, _CHUNK)],
                osems[b]),
        ]
        for c in range(3):
            cps.append(pltpu.make_async_copy(
                col_v.at[pl.ds(b * 16 + 9 + c, 1), pl.ds(0, _CHUNK)],
                out3_t_hbm.at[pl.ds(c, 1), pl.ds(base, _CHUNK)],
                osems[b]))
        return cps

    in_copy(0).start()
    in_copy(1).start()
    in_copy(0).wait()
    in_copy(1).wait()
    _JPH = (_ROWS_PER_W // 2) // _L

    def outer(g, carry):
        for b in range(2):
            ci = g * 2 + b

            @pl.when(g >= 1)
            def _():
                for cp in out_copies(ci - 2, b):
                    cp.wait()

            h = ci * (_CHUNK // _L) // _JPH
            obase = ci * _CHUNK - h * (_ROWS_PER_W // 2)

            @plsc.parallel_loop(0, _CHUNK // _L, 1, unroll=4)
            def row_body(j):
                colv = ids_v[h, pl.ds(obase + j * _L, _L)]
                ids = (colv + 0.5).astype(jnp.int32)
                for c in range(TABLE_COLS):
                    v = plsc.load_gather(table_v, [ids + (c * NUM_VIEWS)])
                    col_v[b * 16 + c, pl.ds(j * _L, _L)] = v

            for cp in out_copies(ci, b):
                cp.start()
        return carry

    lax.fori_loop(0, _N_CHUNKS // 2, outer, 0)
    for b in range(2):
        for cp in out_copies(_N_CHUNKS - 2 + b, b):
            cp.wait()


def kernel(rays, color_embedding):
    mesh = plsc.VectorSubcoreMesh(
        core_axis_name="c", subcore_axis_name="s",
        num_cores=_NC, num_subcores=_NS)
    f = pl.kernel(
        _sc_body,
        out_type=(
            jax.ShapeDtypeStruct((9, N_RAYS), jnp.float32),
            jax.ShapeDtypeStruct((3, N_RAYS), jnp.float32),
        ),
        mesh=mesh,
        compiler_params=pltpu.CompilerParams(needs_layout_passes=False),
        scratch_types=[
            pltpu.VMEM((2, _ROWS_PER_W // 2), jnp.float32),
            pltpu.VMEM((NUM_VIEWS * TABLE_COLS,), jnp.float32),
            pltpu.VMEM((2 * 16, _CHUNK), jnp.float32),
            pltpu.SemaphoreType.DMA,
            pltpu.SemaphoreType.DMA,
            pltpu.SemaphoreType.DMA,
            pltpu.SemaphoreType.DMA,
        ],
    )
    table_flat = color_embedding.T.reshape(NUM_VIEWS * TABLE_COLS)
    out9_t, out3_t = f(rays.T, table_flat)
    return out9_t.T, out3_t.T


# speedup vs baseline: 1.0147x; 1.0147x over previous
import jax
import jax.numpy as jnp
from jax import lax
from jax.experimental import pallas as pl
from jax.experimental.pallas import tpu as pltpu
from jax.experimental.pallas import tpu_sc as plsc

N_RAYS = 1048576
RAY_DIM = 16
NUM_VIEWS = 1000
TABLE_COLS = 12

_NC, _NS, _L = 2, 16, 16
_NW = _NC * _NS
_ROWS_PER_W = N_RAYS // _NW
_CHUNK = 2048
_N_CHUNKS = _ROWS_PER_W // _CHUNK


def _sc_body(rays_t_hbm, table_hbm, out9_t_hbm, out3_t_hbm,
             ids_v, table_v, col_v, isem0, isem1, osem0, osem1):
    wid = lax.axis_index("s") * _NC + lax.axis_index("c")
    base_w = wid * _ROWS_PER_W
    isems = (isem0, isem1)
    osems = (osem0, osem1)
    pltpu.sync_copy(table_hbm, table_v)

    def in_copy(ci, b):
        return pltpu.make_async_copy(
            rays_t_hbm.at[pl.ds(RAY_DIM - 2, 1),
                          pl.ds(base_w + ci * _CHUNK, _CHUNK)],
            ids_v.at[pl.ds(b, 1), pl.ds(0, _CHUNK)],
            isems[b])

    def out_copies(ci, b):
        base = base_w + ci * _CHUNK
        cps = [
            pltpu.make_async_copy(
                col_v.at[pl.ds(b * 16, 8), pl.ds(0, _CHUNK)],
                out9_t_hbm.at[pl.ds(0, 8), pl.ds(base, _CHUNK)],
                osems[b]),
            pltpu.make_async_copy(
                col_v.at[pl.ds(b * 16 + 8, 1), pl.ds(0, _CHUNK)],
                out9_t_hbm.at[pl.ds(8, 1), pl.ds(base, _CHUNK)],
                osems[b]),
        ]
        for c in range(3):
            cps.append(pltpu.make_async_copy(
                col_v.at[pl.ds(b * 16 + 9 + c, 1), pl.ds(0, _CHUNK)],
                out3_t_hbm.at[pl.ds(c, 1), pl.ds(base, _CHUNK)],
                osems[b]))
        return cps

    in_copy(0, 0).start()

    def outer(g, carry):
        for b in range(2):
            ci = g * 2 + b

            @pl.when(ci + 1 < _N_CHUNKS)
            def _():
                in_copy(ci + 1, 1 - b).start()

            in_copy(ci, b).wait()

            @pl.when(g >= 1)
            def _():
                for cp in out_copies(ci - 2, b):
                    cp.wait()

            @plsc.parallel_loop(0, _CHUNK // _L, 1, unroll=2)
            def row_body(j):
                colv = ids_v[b, pl.ds(j * _L, _L)]
                ids = (colv + 0.5).astype(jnp.int32)
                for c in range(TABLE_COLS):
                    v = plsc.load_gather(table_v, [ids + (c * NUM_VIEWS)])
                    col_v[b * 16 + c, pl.ds(j * _L, _L)] = v

            for cp in out_copies(ci, b):
                cp.start()
        return carry

    lax.fori_loop(0, _N_CHUNKS // 2, outer, 0)
    for b in range(2):
        for cp in out_copies(_N_CHUNKS - 2 + b, b):
            cp.wait()


def kernel(rays, color_embedding):
    mesh = plsc.VectorSubcoreMesh(
        core_axis_name="c", subcore_axis_name="s",
        num_cores=_NC, num_subcores=_NS)
    f = pl.kernel(
        _sc_body,
        out_type=(
            jax.ShapeDtypeStruct((9, N_RAYS), jnp.float32),
            jax.ShapeDtypeStruct((3, N_RAYS), jnp.float32),
        ),
        mesh=mesh,
        compiler_params=pltpu.CompilerParams(needs_layout_passes=False),
        scratch_types=[
            pltpu.VMEM((2, _CHUNK), jnp.float32),
            pltpu.VMEM((NUM_VIEWS * TABLE_COLS,), jnp.float32),
            pltpu.VMEM((2 * 16, _CHUNK), jnp.float32),
            pltpu.SemaphoreType.DMA,
            pltpu.SemaphoreType.DMA,
            pltpu.SemaphoreType.DMA,
            pltpu.SemaphoreType.DMA,
        ],
    )
    table_flat = color_embedding.T.reshape(NUM_VIEWS * TABLE_COLS)
    out9_t, out3_t = f(rays.T, table_flat)
    return out9_t.T, out3_t.T
